# baseline (speedup 1.0000x reference)
import jax
import jax.numpy as jnp
from jax.experimental import pallas as pl


def _add_body(x_ref, pe_ref, o_ref):
    o_ref[...] = x_ref[...] + pe_ref[...][None, :, :]


def kernel(x, pe):
    B, S, D = x.shape
    R = 2048
    return pl.pallas_call(
        _add_body,
        grid=(S // R, B),
        in_specs=[
            pl.BlockSpec((1, R, D), lambda i, b: (b, i, 0)),
            pl.BlockSpec((R, D), lambda i, b: (i, 0)),
        ],
        out_specs=pl.BlockSpec((1, R, D), lambda i, b: (b, i, 0)),
        out_shape=jax.ShapeDtypeStruct(x.shape, x.dtype),
    )(x, pe)
